# baseline (speedup 1.0000x reference)
import jax
import jax.numpy as jnp
from jax import lax
from jax.experimental import pallas as pl
from jax.experimental.pallas import tpu as pltpu
from jax.experimental.pallas import tpu_sc as plsc

_BATCH = 64
_SENT = 512
_HID = 768
_SEGS = 2

_N = _BATCH * _SENT
_NW = 32
_PER_W = _N // _NW
_SPW = _SENT // _NW
_W = 32
_CHUNKS = _PER_W // _W
_LANES = 16
_HSL = _HID // _LANES


def _emb_kernel(seqt_hbm, segt_hbm, oidx_hbm, tok_tab, psg_tab, out_hbm,
                idx_v, sidx_v, oidx_v, psg_v, tok0, tok1,
                st0, st1, so0, so1):
    wid = lax.axis_index("s") * 2 + lax.axis_index("c")
    base = wid * _PER_W
    sbase = wid * _SPW

    toks = (tok0, tok1)
    sts = (st0, st1)
    sos = (so0, so1)

    pltpu.sync_copy(seqt_hbm.at[pl.ds(base, _PER_W)], idx_v)
    pltpu.sync_copy(segt_hbm.at[pl.ds(base, _PER_W)], sidx_v)
    pltpu.sync_copy(oidx_hbm.at[wid], oidx_v)
    pltpu.sync_copy(psg_tab.at[pl.ds(sbase, _SPW)], psg_v.at[pl.ds(0, _SPW)])
    pltpu.sync_copy(psg_tab.at[pl.ds(_SENT + sbase, _SPW)],
                    psg_v.at[pl.ds(_SPW, _SPW)])

    def gather(b, coff):
        return pltpu.make_async_copy(
            tok_tab.at[idx_v.at[pl.ds(coff, _W)]], toks[b], sts[b])

    def out_scatter(b, chunk):
        return pltpu.make_async_copy(
            toks[b], out_hbm.at[oidx_v.at[chunk]], sos[b])

    for b in range(2):
        gather(b, b * _W).start()

    def step(it, _):
        for b in range(2):
            chunk = 2 * it + b
            coff = chunk * _W
            gather(b, coff).wait()

            def add_group(jg, _):
                sv = sidx_v[pl.ds(coff + jg * _LANES, _LANES)]
                for jl in range(_LANES):
                    row = sv[jl] * _SPW + it
                    j = jg * _LANES + jl
                    for h in range(_HSL):
                        sl = pl.ds(h * _LANES, _LANES)
                        plsc.addupdate(toks[b].at[j, sl], psg_v[row, sl])
                return 0

            lax.fori_loop(0, _W // _LANES, add_group, 0)

            out_scatter(b, chunk).start()

            @pl.when(chunk + 2 < _CHUNKS)
            def _():
                out_scatter(b, chunk).wait()
                gather(b, coff + 2 * _W).start()

        return 0

    lax.fori_loop(0, _CHUNKS // 2, step, 0)

    for b in range(2):
        out_scatter(b, _CHUNKS - 2 + b).wait()


@jax.jit
def _emb(seqt_flat, segt_flat, oidx, token_table, psg_table):
    mesh = plsc.VectorSubcoreMesh(core_axis_name="c", subcore_axis_name="s")
    kfn = pl.kernel(
        _emb_kernel,
        out_type=jax.ShapeDtypeStruct((_N, _HID), jnp.float32),
        mesh=mesh,
        scratch_types=[
            pltpu.VMEM((_PER_W,), jnp.int32),
            pltpu.VMEM((_PER_W,), jnp.int32),
            pltpu.VMEM((_CHUNKS, _W), jnp.int32),
            pltpu.VMEM((_SEGS * _SPW, _HID), jnp.float32),
            pltpu.VMEM((_W, _HID), jnp.float32),
            pltpu.VMEM((_W, _HID), jnp.float32),
            pltpu.SemaphoreType.DMA,
            pltpu.SemaphoreType.DMA,
            pltpu.SemaphoreType.DMA,
            pltpu.SemaphoreType.DMA,
        ],
    )
    return kfn(seqt_flat, segt_flat, oidx, token_table, psg_table)


def kernel(seq, seg, token_table, position_table, segment_table):
    seqt_flat = seq.T.reshape(-1).astype(jnp.int32)
    segt_flat = seg.T.reshape(-1).astype(jnp.int32)
    psg_table = (segment_table[:, None, :] + position_table[None, :, :]
                 ).reshape(_SEGS * _SENT, _HID)
    s_ids = jnp.arange(_SENT, dtype=jnp.int32)
    b_ids = jnp.arange(_BATCH, dtype=jnp.int32)
    oidx = (b_ids[None, :] * _SENT + s_ids[:, None]).reshape(_NW, _CHUNKS, _W)
    out = _emb(seqt_flat, segt_flat, oidx, token_table, psg_table)
    return out.reshape(_BATCH, _SENT, _HID)


# speedup vs baseline: 1.9105x; 1.9105x over previous
import jax
import jax.numpy as jnp
from jax import lax
from jax.experimental import pallas as pl
from jax.experimental.pallas import tpu as pltpu
from jax.experimental.pallas import tpu_sc as plsc

_BATCH = 64
_SENT = 512
_HID = 768
_SEGS = 2

_N = _BATCH * _SENT
_NW = 32
_PER_W = _N // _NW
_W = 16
_S = 3
_CHUNKS = _PER_W // _W
_LANES = 16
_HSL = _HID // _LANES


def _emb_kernel(seq_hbm, psg_idx_hbm, tok_tab, psg_tab, out_hbm,
                idx_v, pidx_v, *bufs_and_sems):
    toks = bufs_and_sems[0:_S]
    accs = bufs_and_sems[_S:2 * _S]
    sts = bufs_and_sems[2 * _S:3 * _S]
    sps = bufs_and_sems[3 * _S:4 * _S]
    sos = bufs_and_sems[4 * _S:5 * _S]

    wid = lax.axis_index("s") * 2 + lax.axis_index("c")
    base = wid * _PER_W

    pltpu.sync_copy(seq_hbm.at[pl.ds(base, _PER_W)], idx_v)
    pltpu.sync_copy(psg_idx_hbm.at[pl.ds(base, _PER_W)], pidx_v)

    def gathers(b, coff):
        t = pltpu.make_async_copy(
            tok_tab.at[idx_v.at[pl.ds(coff, _W)]], toks[b], sts[b])
        p = pltpu.make_async_copy(
            psg_tab.at[pidx_v.at[pl.ds(coff, _W)]], accs[b], sps[b])
        return t, p

    def out_copy(b, coff):
        return pltpu.make_async_copy(
            accs[b], out_hbm.at[pl.ds(base + coff, _W)], sos[b])

    for b in range(_S):
        t, p = gathers(b, b * _W)
        t.start()
        p.start()

    def chunk_body(chunk, b, first):
        coff = chunk * _W
        t, p = gathers(b, coff)
        t.wait()
        p.wait()

        def add_row(j, _):
            for h in range(_HSL):
                sl = pl.ds(h * _LANES, _LANES)
                plsc.addupdate(accs[b].at[j, sl], toks[b][j, sl])
            return 0

        lax.fori_loop(0, _W, add_row, 0, unroll=2)

        out_copy(b, coff).start()

        pb = (b - 1) % _S
        pcoff = coff - _W

        def drain_and_refill():
            out_copy(pb, pcoff).wait()

            @pl.when(pcoff + _S * _W < _CHUNKS * _W)
            def _():
                t2, p2 = gathers(pb, pcoff + _S * _W)
                t2.start()
                p2.start()

        if first:
            @pl.when(chunk >= 1)
            def _():
                drain_and_refill()
        else:
            drain_and_refill()

    def step(it, _):
        for b in range(_S):
            chunk_body(_S * it + b, b, first=(b == 0))
        return 0

    lax.fori_loop(0, _CHUNKS // _S, step, 0)

    done = (_CHUNKS // _S) * _S
    for chunk in range(done, _CHUNKS):
        chunk_body(chunk, chunk % _S, first=False)

    out_copy((_CHUNKS - 1) % _S, (_CHUNKS - 1) * _W).wait()


@jax.jit
def _emb(seq_flat, psg_idx, token_table, psg_table):
    mesh = plsc.VectorSubcoreMesh(core_axis_name="c", subcore_axis_name="s")
    scratch = (
        [pltpu.VMEM((_PER_W,), jnp.int32)] * 2
        + [pltpu.VMEM((_W, _HID), jnp.float32)] * (2 * _S)
        + [pltpu.SemaphoreType.DMA] * (3 * _S)
    )
    kfn = pl.kernel(
        _emb_kernel,
        out_type=jax.ShapeDtypeStruct((_N, _HID), jnp.float32),
        mesh=mesh,
        scratch_types=scratch,
    )
    return kfn(seq_flat, psg_idx, token_table, psg_table)


def kernel(seq, seg, token_table, position_table, segment_table):
    seq_flat = seq.reshape(-1).astype(jnp.int32)
    seg_flat = seg.reshape(-1).astype(jnp.int32)
    psg_table = (segment_table[:, None, :] + position_table[None, :, :]
                 ).reshape(_SEGS * _SENT, _HID)
    pos_flat = jnp.tile(jnp.arange(_SENT, dtype=jnp.int32), _BATCH)
    psg_idx = seg_flat * _SENT + pos_flat
    out = _emb(seq_flat, psg_idx, token_table, psg_table)
    return out.reshape(_BATCH, _SENT, _HID)
